# baseline (speedup 1.0000x reference)
import functools

import jax
import jax.numpy as jnp
from jax import lax
from jax.experimental import pallas as pl
from jax.experimental.pallas import tpu as pltpu
from jax.experimental.pallas import tpu_sc as plsc

_NF = 26
_V = 100000
_D = 32
_B = 16384
_OD = _NF * _D
_EPS = 1e-5

_NW = 32
_RPW = _B // _NW
_CH = 8
_NCHUNK = _RPW // _CH
_IPC = _CH * _NF
_NVEC = _IPC // 16
_NG = 2

_GDN = lax.GatherDimensionNumbers(
    offset_dims=(), collapsed_slice_dims=(0,), start_index_map=(0,))


def _shuf(x, perm):
    return lax.gather(x, perm, _GDN, slice_sizes=(1,),
                      mode=lax.GatherScatterMode.PROMISE_IN_BOUNDS)


def _allsum(x, perms):
    for p in perms:
        x = x + _shuf(x, p)
    return x


def _make_sc_kernel():
    mesh = plsc.VectorSubcoreMesh(core_axis_name="c", subcore_axis_name="s")

    @functools.partial(
        pl.kernel,
        mesh=mesh,
        out_type=jax.ShapeDtypeStruct((_B, _OD), jnp.float32),
        scratch_types=[
            pltpu.VMEM((_NCHUNK, _IPC), jnp.int32),
            pltpu.VMEM((_IPC,), jnp.int32),
            pltpu.VMEM((_NG, 128), jnp.int32),
            pltpu.VMEM((_NG, 128), jnp.int32),
            pltpu.VMEM((_NG * 128, 128), jnp.float32),
            pltpu.VMEM((_NG * 128, 128), jnp.float32),
            pltpu.VMEM((_CH, _OD), jnp.float32),
            pltpu.VMEM((_OD,), jnp.float32),
            pltpu.VMEM((_OD,), jnp.float32),
            pltpu.SemaphoreType.DMA,
        ],
    )
    def emb_ln(cat_hbm, tab_hbm, enc_hbm, g_hbm, b_hbm, out_hbm,
               catb, offb, idxb, sidxb, rowb, subx, outb, gb, bb, sem):
        wid = lax.axis_index("s") * 2 + lax.axis_index("c")
        pltpu.sync_copy(g_hbm, gb)
        pltpu.sync_copy(b_hbm, bb)
        pltpu.sync_copy(
            cat_hbm.at[pl.ds(pl.multiple_of(wid * _NCHUNK, _NCHUNK), _NCHUNK)],
            catb)
        lanes = lax.iota(jnp.int32, 16)
        perms = [(lanes ^ k)[:, None] for k in (8, 4, 2, 1)]
        zeros = jnp.zeros((16,), jnp.int32)
        for v in range(_NVEC):
            offb[pl.ds(v * 16, 16)] = ((v * 16 + lanes) % _NF) * _V
        for v in range(_NVEC, _NG * 8):
            idxb[v // 8, pl.ds((v % 8) * 16, 16)] = zeros
            sidxb[v // 8, pl.ds((v % 8) * 16, 16)] = zeros

        def chunk_body(c, carry):
            for v in range(_NVEC):
                cv = catb[c, pl.ds(v * 16, 16)]
                cv = jnp.minimum(jnp.maximum(cv, 0), _V - 1)
                flat = cv + offb[pl.ds(v * 16, 16)]
                idxb[v // 8, pl.ds((v % 8) * 16, 16)] = flat >> 2
                sidxb[v // 8, pl.ds((v % 8) * 16, 16)] = flat & 3
            cps = [
                pltpu.async_copy(tab_hbm.at[idxb.at[g]],
                                 rowb.at[pl.ds(g * 128, 128)], sem)
                for g in range(_NG)
            ] + [
                pltpu.async_copy(enc_hbm.at[sidxb.at[g]],
                                 subx.at[pl.ds(g * 128, 128)], sem)
                for g in range(_NG)
            ]
            for cp in cps:
                cp.wait()

            def row_body(r, rcarry):
                p0 = r * _NF
                s = jnp.zeros((16,), jnp.float32)
                q = jnp.zeros((16,), jnp.float32)
                for f in range(_NF):
                    p = p0 + f
                    sv = subx[p, pl.ds(0, 16)]
                    m0 = sv < 0.5
                    m1 = sv < 1.5
                    m2 = sv < 2.5
                    for h in range(2):
                        v0 = rowb[p, pl.ds(h * 16, 16)]
                        v1 = rowb[p, pl.ds(32 + h * 16, 16)]
                        v2 = rowb[p, pl.ds(64 + h * 16, 16)]
                        v3 = rowb[p, pl.ds(96 + h * 16, 16)]
                        vv = jnp.where(m0, v0,
                                       jnp.where(m1, v1, jnp.where(m2, v2, v3)))
                        outb[r, pl.ds(f * _D + h * 16, 16)] = vv
                        s = s + vv
                        q = q + vv * vv
                meanv = _allsum(s, perms) * (1.0 / _OD)
                xv = _allsum(q, perms) * (1.0 / _OD) - meanv * meanv + _EPS
                iv = 0x5F3759DF - (lax.bitcast_convert_type(xv, jnp.int32) >> 1)
                y = lax.bitcast_convert_type(iv, jnp.float32)
                for _ in range(3):
                    y = y * (1.5 - 0.5 * xv * y * y)
                for f in range(_NF):
                    for h in range(2):
                        col = f * _D + h * 16
                        vv = outb[r, pl.ds(col, 16)]
                        gv = gb[pl.ds(col, 16)]
                        bv = bb[pl.ds(col, 16)]
                        outb[r, pl.ds(col, 16)] = (vv - meanv) * y * gv + bv
                return rcarry

            lax.fori_loop(0, _CH, row_body, 0)
            row0 = pl.multiple_of(wid * _RPW + c * _CH, _CH)
            pltpu.sync_copy(outb, out_hbm.at[pl.ds(row0, _CH)])
            return carry

        lax.fori_loop(0, _NCHUNK, chunk_body, 0)

    return emb_ln


_EMB_LN = _make_sc_kernel()


def kernel(cat, tables, gamma, beta):
    tab = tables.reshape(_NF * _V // 4, _D * 4)
    catv = cat.reshape(_B // _CH, _IPC)
    enc = jnp.broadcast_to(
        jnp.arange(8, dtype=jnp.float32)[:, None], (8, 128))
    return _EMB_LN(catv, tab, enc, gamma, beta)


# speedup vs baseline: 6.0635x; 6.0635x over previous
import functools

import jax
import jax.numpy as jnp
from jax import lax
from jax.experimental import pallas as pl
from jax.experimental.pallas import tpu as pltpu
from jax.experimental.pallas import tpu_sc as plsc

_NF = 26
_V = 100000
_D = 32
_B = 16384
_OD = _NF * _D
_EPS = 1e-5

_NW = 32
_RPW = _B // _NW
_CH = 64
_NCHUNK = _RPW // _CH
_IPC = _CH * _NF
_NVEC = _IPC // 16
_NG = _IPC // 128

_GDN = lax.GatherDimensionNumbers(
    offset_dims=(), collapsed_slice_dims=(0,), start_index_map=(0,))


def _shuf(x, perm):
    return lax.gather(x, perm, _GDN, slice_sizes=(1,),
                      mode=lax.GatherScatterMode.PROMISE_IN_BOUNDS)


def _allsum(x, perms):
    for p in perms:
        x = x + _shuf(x, p)
    return x


def _make_sc_kernel():
    mesh = plsc.VectorSubcoreMesh(core_axis_name="c", subcore_axis_name="s")

    @functools.partial(
        pl.kernel,
        mesh=mesh,
        compiler_params=pltpu.CompilerParams(use_tc_tiling_on_sc=False),
        out_type=jax.ShapeDtypeStruct((_B, _OD), jnp.float32),
        scratch_types=[
            pltpu.VMEM((_IPC,), jnp.int32),
            pltpu.VMEM((_IPC,), jnp.int32),
            pltpu.VMEM((_NG, 128), jnp.int32),
            pltpu.VMEM((_IPC, _D), jnp.float32),
            pltpu.VMEM((_CH, _OD), jnp.float32),
            pltpu.VMEM((_OD,), jnp.float32),
            pltpu.VMEM((_OD,), jnp.float32),
            pltpu.SemaphoreType.DMA,
        ],
    )
    def emb_ln(cat_hbm, tab_hbm, g_hbm, b_hbm, out_hbm,
               catb, offb, idxb, rowb, outb, gb, bb, sem):
        wid = lax.axis_index("s") * 2 + lax.axis_index("c")
        pltpu.sync_copy(g_hbm, gb)
        pltpu.sync_copy(b_hbm, bb)
        lanes = lax.iota(jnp.int32, 16)
        perms = [(lanes ^ k)[:, None] for k in (8, 4, 2, 1)]
        for v in range(_NVEC):
            offb[pl.ds(v * 16, 16)] = ((v * 16 + lanes) % _NF) * _V

        def chunk_body(c, carry):
            row0 = wid * _RPW + c * _CH
            base_e = row0 * _NF
            pltpu.sync_copy(cat_hbm.at[pl.ds(base_e, _IPC)], catb)
            for v in range(_NVEC):
                cv = catb[pl.ds(v * 16, 16)]
                cv = jnp.minimum(jnp.maximum(cv, 0), _V - 1)
                idxb[v // 8, pl.ds((v % 8) * 16, 16)] = cv + offb[pl.ds(v * 16, 16)]
            cps = [
                pltpu.async_copy(tab_hbm.at[idxb.at[g]],
                                 rowb.at[pl.ds(g * 128, 128)], sem)
                for g in range(_NG)
            ]
            for cp in cps:
                cp.wait()

            def row_body(r, rcarry):
                rb = r * _NF
                s = jnp.zeros((16,), jnp.float32)
                q = jnp.zeros((16,), jnp.float32)
                for f in range(_NF):
                    for h in range(2):
                        vv = rowb[rb + f, pl.ds(h * 16, 16)]
                        s = s + vv
                        q = q + vv * vv
                meanv = _allsum(s, perms) * (1.0 / _OD)
                xv = _allsum(q, perms) * (1.0 / _OD) - meanv * meanv + _EPS
                iv = 0x5F3759DF - (lax.bitcast_convert_type(xv, jnp.int32) >> 1)
                y = lax.bitcast_convert_type(iv, jnp.float32)
                for _ in range(3):
                    y = y * (1.5 - 0.5 * xv * y * y)
                for f in range(_NF):
                    for h in range(2):
                        vv = rowb[rb + f, pl.ds(h * 16, 16)]
                        gv = gb[pl.ds(f * _D + h * 16, 16)]
                        bv = bb[pl.ds(f * _D + h * 16, 16)]
                        outb[r, pl.ds(f * _D + h * 16, 16)] = (vv - meanv) * y * gv + bv
                return rcarry

            lax.fori_loop(0, _CH, row_body, 0)
            pltpu.sync_copy(outb, out_hbm.at[pl.ds(row0, _CH)])
            return carry

        lax.fori_loop(0, _NCHUNK, chunk_body, 0)

    return emb_ln


_EMB_LN = _make_sc_kernel()


def kernel(cat, tables, gamma, beta):
    tab = tables.reshape(_NF * _V, _D)
    catf = cat.reshape(-1)
    return _EMB_LN(catf, tab, gamma, beta)
